# baseline (speedup 1.0000x reference)
import jax
import jax.numpy as jnp
from jax import lax
from jax.experimental import pallas as pl
from jax.experimental.pallas import tpu as pltpu
from jax.experimental.pallas import tpu_sc as plsc

N = 10000
E = 320000
D = 128
DH = D // 2
NC = 2
NS = 16
K = 128
CH = 160
EPW = CH * K
EPAD = NS * EPW - E
CHH = CH // 2
NP = 10240
DUMMY = N + 16
RPS = NP // NS
CW = 16
ZR = 32


def _sc_pass(with_counts: bool):
  nb = 4
  out_type = [jax.ShapeDtypeStruct((NC, NP, DH), jnp.float32)]
  scratch = [
      pltpu.VMEM((CH, K), jnp.int32),
      pltpu.VMEM((CH, K), jnp.int32),
      pltpu.VMEM((ZR, DH), jnp.float32),
  ]
  scratch += [pltpu.VMEM((K, DH), jnp.float32) for _ in range(nb)]
  scratch += [pltpu.SemaphoreType.DMA for _ in range(2 * nb)]
  if with_counts:
    out_type.append(jax.ShapeDtypeStruct((NC, NP, CW), jnp.float32))
    scratch += [pltpu.VMEM((K, CW), jnp.float32)]
  scratch += [pltpu.VMEM_SHARED((NP, DH), jnp.float32)]
  if with_counts:
    scratch += [pltpu.VMEM_SHARED((NP, CW), jnp.float32)]

  mesh = plsc.VectorSubcoreMesh(core_axis_name="c", subcore_axis_name="s")

  def body(table, srcs, dsts, ones_h, *rest):
    if with_counts:
      agg_out, cnt_out = rest[0], rest[1]
      rest = rest[2:]
    else:
      agg_out = rest[0]
      rest = rest[1:]
    src_v, dst_v, zbuf = rest[0], rest[1], rest[2]
    bufs = rest[3:3 + nb]
    sem_g = rest[3 + nb:3 + 2 * nb]
    sem_s = rest[3 + 2 * nb:3 + 3 * nb]
    rest = rest[3 + 3 * nb:]
    if with_counts:
      ones_v, agg_sh, cnt_sh = rest
    else:
      (agg_sh,) = rest
    cid = lax.axis_index("c")
    sid = lax.axis_index("s")

    pltpu.sync_copy(srcs.at[cid * NS + sid], src_v)
    pltpu.sync_copy(dsts.at[sid], dst_v)
    if with_counts:
      pltpu.sync_copy(ones_h, ones_v)
    z16 = jnp.zeros((16,), jnp.float32)

    def zrow(r, _):
      for c in range(DH // 16):
        zbuf[r, pl.ds(c * 16, 16)] = z16
      return 0

    lax.fori_loop(0, ZR, zrow, 0)
    for r in range(RPS // ZR):
      pltpu.sync_copy(zbuf, agg_sh.at[pl.ds(sid * RPS + r * ZR, ZR)])
    if with_counts:
      for r in range(RPS // ZR):
        pltpu.sync_copy(zbuf.at[:, pl.ds(0, CW)],
                        cnt_sh.at[pl.ds(sid * RPS + r * ZR, ZR)])
    plsc.subcore_barrier()

    def gather(jj, b):
      return pltpu.async_copy(table.at[src_v.at[jj]], bufs[b], sem_g[b])

    def scatter(jj, b):
      return pltpu.async_copy(bufs[b], agg_sh.at[dst_v.at[jj]], sem_s[b],
                              add=True)

    for b in range(nb):
      gather(b, b)

    def step(j, _):
      for b in range(nb):
        jj = j * nb + b
        pltpu.make_async_copy(table.at[src_v.at[jj]], bufs[b],
                              sem_g[b]).wait()
        scatter(jj, b)
        if with_counts:
          @pl.when(lax.select(cid == 0, jj < CHH, jj >= CHH))
          def _():
            pltpu.sync_copy(ones_v, cnt_sh.at[dst_v.at[jj]], add=True)
        bp = (b + nb - 1) % nb

        @pl.when(jnp.logical_and(jj >= 1, jj + nb - 1 < CH))
        def _():
          pltpu.make_async_copy(bufs[bp], agg_sh.at[dst_v.at[jj - 1]],
                                sem_s[bp]).wait()
          gather(jj + nb - 1, bp)
      return 0

    lax.fori_loop(0, CH // nb, step, 0)
    for b in range(nb):
      m = CH - nb + b
      pltpu.make_async_copy(bufs[m % nb], agg_sh.at[dst_v.at[m]],
                            sem_s[m % nb]).wait()
    plsc.subcore_barrier()

    row_sl = pl.ds(sid * RPS, RPS)
    pltpu.sync_copy(agg_sh.at[row_sl], agg_out.at[cid].at[row_sl])
    if with_counts:
      pltpu.sync_copy(cnt_sh.at[row_sl], cnt_out.at[cid].at[row_sl])

  return pl.kernel(body, out_type=tuple(out_type), mesh=mesh,
                   scratch_types=scratch,
                   compiler_params=pltpu.CompilerParams(
                       use_tc_tiling_on_sc=False))


_sc_agg_counts = _sc_pass(with_counts=True)
_sc_agg = _sc_pass(with_counts=False)


def _tc_combine(relu: bool):
  BLK = 2000
  GRID = N // BLK

  def body(aggp, cntp, x, wlt, wrt, b, o):
    cnt = cntp[0, :, 0:1] + cntp[1, :, 0:1]
    recip = 1.0 / jnp.maximum(cnt, 1.0)
    y = (jnp.dot(aggp[0] * recip, wlt[0:DH, :],
                 preferred_element_type=jnp.float32)
         + jnp.dot(aggp[1] * recip, wlt[DH:D, :],
                   preferred_element_type=jnp.float32)
         + jnp.dot(x[...], wrt[...], preferred_element_type=jnp.float32)
         + b[...])
    o[...] = jnp.maximum(y, 0.0) if relu else y

  return pl.pallas_call(
      body,
      grid=(GRID,),
      in_specs=[
          pl.BlockSpec((NC, BLK, DH), lambda i: (0, i, 0)),
          pl.BlockSpec((NC, BLK, CW), lambda i: (0, i, 0)),
          pl.BlockSpec((BLK, D), lambda i: (i, 0)),
          pl.BlockSpec((D, D), lambda i: (0, 0)),
          pl.BlockSpec((D, D), lambda i: (0, 0)),
          pl.BlockSpec((1, D), lambda i: (0, 0)),
      ],
      out_specs=pl.BlockSpec((BLK, D), lambda i: (i, 0)),
      out_shape=jax.ShapeDtypeStruct((N, D), jnp.float32),
  )


_tc_combine1 = _tc_combine(relu=True)
_tc_combine2 = _tc_combine(relu=False)


def kernel(x, edge_index, W1l, b1, W1r, W2l, b2, W2r):
  src1 = jnp.concatenate([edge_index[0], jnp.zeros((EPAD,), jnp.int32)])
  src = jnp.concatenate([src1 * 2, src1 * 2 + 1]).reshape(NC * NS, CH, K)
  dst = jnp.concatenate(
      [edge_index[1], jnp.full((EPAD,), DUMMY, jnp.int32)]).reshape(NS, CH, K)
  ones_h = jnp.ones((K, CW), jnp.float32)

  agg1, cnt = _sc_agg_counts(x.reshape(NC * N, DH), src, dst, ones_h)
  h = _tc_combine1(agg1, cnt, x, W1l.T, W1r.T, b1.reshape(1, D))
  (agg2,) = _sc_agg(h.reshape(NC * N, DH), src, dst, ones_h)
  out = _tc_combine2(agg2, cnt, h, W2l.T, W2r.T, b2.reshape(1, D))
  return out


# speedup vs baseline: 2.9310x; 2.9310x over previous
import jax
import jax.numpy as jnp
from jax import lax
from jax.experimental import pallas as pl
from jax.experimental.pallas import tpu as pltpu
from jax.experimental.pallas import tpu_sc as plsc

N = 10000
E = 320000
D = 128
DH = D // 2
NC = 2
NS = 16
K = 128
CH = 160
EPW = CH * K
EPAD = NS * EPW - E
CHH = CH // 2
NP = 10240
DUMMY = N + 16
RPS = NP // NS
CW = 16
ZR = 32


def _sc_pass(with_counts: bool):
  nb = 4
  out_type = [jax.ShapeDtypeStruct((NC, NP, DH), jnp.float32)]
  scratch = [
      pltpu.VMEM((CH, K), jnp.int32),
      pltpu.VMEM((CH, K), jnp.int32),
      pltpu.VMEM((ZR, DH), jnp.float32),
  ]
  scratch += [pltpu.VMEM((K, DH), jnp.float32) for _ in range(nb)]
  scratch += [pltpu.SemaphoreType.DMA for _ in range(2 * nb)]
  if with_counts:
    out_type.append(jax.ShapeDtypeStruct((NC, NP, CW), jnp.float32))
    scratch += [pltpu.VMEM((K, CW), jnp.float32)]
  scratch += [pltpu.VMEM_SHARED((NP, DH), jnp.float32)]
  if with_counts:
    scratch += [pltpu.VMEM_SHARED((NP, CW), jnp.float32)]

  mesh = plsc.VectorSubcoreMesh(core_axis_name="c", subcore_axis_name="s")

  def body(table, srcs, dsts, ones_h, *rest):
    if with_counts:
      agg_out, cnt_out = rest[0], rest[1]
      rest = rest[2:]
    else:
      agg_out = rest[0]
      rest = rest[1:]
    src_v, dst_v, zbuf = rest[0], rest[1], rest[2]
    bufs = rest[3:3 + nb]
    sem_g = rest[3 + nb:3 + 2 * nb]
    sem_s = rest[3 + 2 * nb:3 + 3 * nb]
    rest = rest[3 + 3 * nb:]
    if with_counts:
      ones_v, agg_sh, cnt_sh = rest
    else:
      (agg_sh,) = rest
    cid = lax.axis_index("c")
    sid = lax.axis_index("s")

    pltpu.sync_copy(srcs.at[cid * NS + sid], src_v)
    pltpu.sync_copy(dsts.at[sid], dst_v)
    if with_counts:
      pltpu.sync_copy(ones_h, ones_v)
    z16 = jnp.zeros((16,), jnp.float32)

    def zrow(r, _):
      for c in range(DH // 16):
        zbuf[r, pl.ds(c * 16, 16)] = z16
      return 0

    lax.fori_loop(0, ZR, zrow, 0)
    for r in range(RPS // ZR):
      pltpu.sync_copy(zbuf, agg_sh.at[pl.ds(sid * RPS + r * ZR, ZR)])
    if with_counts:
      for r in range(RPS // ZR):
        pltpu.sync_copy(zbuf.at[:, pl.ds(0, CW)],
                        cnt_sh.at[pl.ds(sid * RPS + r * ZR, ZR)])
    plsc.subcore_barrier()

    def gather(jj, b):
      return pltpu.async_copy(table.at[src_v.at[jj]], bufs[b], sem_g[b])

    def scatter(jj, b):
      return pltpu.async_copy(bufs[b], agg_sh.at[dst_v.at[jj]], sem_s[b],
                              add=True)

    for b in range(nb):
      gather(b, b)

    def step(j, _):
      for b in range(nb):
        jj = j * nb + b
        pltpu.make_async_copy(table.at[src_v.at[jj]], bufs[b],
                              sem_g[b]).wait()
        scatter(jj, b)
        if with_counts:
          @pl.when(lax.select(cid == 0, jj < CHH, jj >= CHH))
          def _():
            pltpu.sync_copy(ones_v, cnt_sh.at[dst_v.at[jj]], add=True)
        bp = (b + nb - 1) % nb

        @pl.when(jnp.logical_and(jj >= 1, jj + nb - 1 < CH))
        def _():
          pltpu.make_async_copy(bufs[bp], agg_sh.at[dst_v.at[jj - 1]],
                                sem_s[bp]).wait()
          gather(jj + nb - 1, bp)
      return 0

    lax.fori_loop(0, CH // nb, step, 0)
    for b in range(nb):
      m = CH - nb + b
      pltpu.make_async_copy(bufs[m % nb], agg_sh.at[dst_v.at[m]],
                            sem_s[m % nb]).wait()
    plsc.subcore_barrier()

    row_sl = pl.ds(sid * RPS, RPS)
    pltpu.sync_copy(agg_sh.at[row_sl], agg_out.at[cid].at[row_sl])
    if with_counts:
      pltpu.sync_copy(cnt_sh.at[row_sl], cnt_out.at[cid].at[row_sl])

  return pl.kernel(body, out_type=tuple(out_type), mesh=mesh,
                   scratch_types=scratch,
                   compiler_params=pltpu.CompilerParams(
                       use_tc_tiling_on_sc=False))


_sc_agg_counts = _sc_pass(with_counts=True)
_sc_agg = _sc_pass(with_counts=False)


def _tc_combine(relu: bool):
  BLK = 2000
  GRID = N // BLK

  def body(aggp, cntp, x, wlt, wrt, b, o):
    cnt = cntp[0, :, 0:1] + cntp[1, :, 0:1]
    recip = 1.0 / jnp.maximum(cnt, 1.0)
    y = (jnp.dot(aggp[0] * recip, wlt[0:DH, :],
                 preferred_element_type=jnp.float32)
         + jnp.dot(aggp[1] * recip, wlt[DH:D, :],
                   preferred_element_type=jnp.float32)
         + jnp.dot(x[...], wrt[...], preferred_element_type=jnp.float32)
         + b[...])
    o[...] = jnp.maximum(y, 0.0) if relu else y

  return pl.pallas_call(
      body,
      grid=(GRID,),
      in_specs=[
          pl.BlockSpec((NC, BLK, DH), lambda i: (0, i, 0)),
          pl.BlockSpec((NC, BLK, CW), lambda i: (0, i, 0)),
          pl.BlockSpec((BLK, D), lambda i: (i, 0)),
          pl.BlockSpec((D, D), lambda i: (0, 0)),
          pl.BlockSpec((D, D), lambda i: (0, 0)),
          pl.BlockSpec((1, D), lambda i: (0, 0)),
      ],
      out_specs=pl.BlockSpec((BLK, D), lambda i: (i, 0)),
      out_shape=jax.ShapeDtypeStruct((N, D), jnp.float32),
  )


_tc_combine1 = _tc_combine(relu=True)
_tc_combine2 = _tc_combine(relu=False)


def kernel(x, edge_index, W1l, b1, W1r, W2l, b2, W2r):
  pad_src = jnp.arange(EPAD, dtype=jnp.int32) % N
  src1 = jnp.concatenate([edge_index[0], pad_src])
  src = jnp.concatenate([src1 * 2, src1 * 2 + 1]).reshape(NC * NS, CH, K)
  pad_dst = N + (jnp.arange(EPAD, dtype=jnp.int32) % (NP - N))
  dst = jnp.concatenate([edge_index[1], pad_dst]).reshape(NS, CH, K)
  ones_h = jnp.ones((K, CW), jnp.float32)

  agg1, cnt = _sc_agg_counts(x.reshape(NC * N, DH), src, dst, ones_h)
  h = _tc_combine1(agg1, cnt, x, W1l.T, W1r.T, b1.reshape(1, D))
  (agg2,) = _sc_agg(h.reshape(NC * N, DH), src, dst, ones_h)
  out = _tc_combine2(agg2, cnt, h, W2l.T, W2r.T, b2.reshape(1, D))
  return out


# speedup vs baseline: 3.1629x; 1.0791x over previous
import jax
import jax.numpy as jnp
from jax import lax
from jax.experimental import pallas as pl
from jax.experimental.pallas import tpu as pltpu
from jax.experimental.pallas import tpu_sc as plsc

N = 10000
E = 320000
D = 128
DH = D // 2
NC = 2
NS = 16
K = 128
CH = 160
EPW = CH * K
EPAD = NS * EPW - E
CHH = CH // 2
NP = 10240
DUMMY = N + 16
RPS = NP // NS
CW = 16


def _sc_pass(with_counts: bool):
  nb = 4
  out_type = [jax.ShapeDtypeStruct((NP, D), jnp.float32)]
  scratch = [
      pltpu.VMEM((CH, K), jnp.int32),
      pltpu.VMEM((CH, K), jnp.int32),
  ]
  scratch += [pltpu.VMEM((K, DH), jnp.float32) for _ in range(nb)]
  scratch += [pltpu.SemaphoreType.DMA for _ in range(2 * nb)]
  if with_counts:
    out_type.append(jax.ShapeDtypeStruct((NC, NP, CW), jnp.float32))
    scratch += [pltpu.VMEM((K, CW), jnp.float32)]
  scratch += [pltpu.VMEM_SHARED((NP, DH), jnp.float32)]
  if with_counts:
    scratch += [pltpu.VMEM_SHARED((NP, CW), jnp.float32)]

  mesh = plsc.VectorSubcoreMesh(core_axis_name="c", subcore_axis_name="s")

  def body(table, srcs, dsts, ones_h, *rest):
    if with_counts:
      agg_out, cnt_out = rest[0], rest[1]
      rest = rest[2:]
    else:
      agg_out = rest[0]
      rest = rest[1:]
    src_v, dst_v = rest[0], rest[1]
    bufs = rest[2:2 + nb]
    sem_g = rest[2 + nb:2 + 2 * nb]
    sem_s = rest[2 + 2 * nb:2 + 3 * nb]
    rest = rest[2 + 3 * nb:]
    if with_counts:
      ones_v, agg_sh, cnt_sh = rest
    else:
      (agg_sh,) = rest
    cid = lax.axis_index("c")
    sid = lax.axis_index("s")

    pltpu.sync_copy(srcs.at[cid * NS + sid], src_v)
    pltpu.sync_copy(dsts.at[sid], dst_v)
    if with_counts:
      pltpu.sync_copy(ones_h, ones_v)
    z16 = jnp.zeros((16,), jnp.float32)

    def zrow(r, _):
      for c in range(DH // 16):
        bufs[0][r, pl.ds(c * 16, 16)] = z16
      return 0

    lax.fori_loop(0, K, zrow, 0)
    for r in range(RPS // K):
      pltpu.sync_copy(bufs[0], agg_sh.at[pl.ds(sid * RPS + r * K, K)])
    if with_counts:
      for r in range(RPS // K):
        pltpu.sync_copy(bufs[0].at[:, pl.ds(0, CW)],
                        cnt_sh.at[pl.ds(sid * RPS + r * K, K)])
    plsc.subcore_barrier()

    def gather(jj, b):
      return pltpu.async_copy(table.at[src_v.at[jj]], bufs[b], sem_g[b])

    def scatter(jj, b):
      return pltpu.async_copy(bufs[b], agg_sh.at[dst_v.at[jj]], sem_s[b],
                              add=True)

    for b in range(nb):
      gather(b, b)

    def step(j, _):
      for b in range(nb):
        jj = j * nb + b
        pltpu.make_async_copy(table.at[src_v.at[jj]], bufs[b],
                              sem_g[b]).wait()
        scatter(jj, b)
        if with_counts:
          @pl.when(lax.select(cid == 0, jj < CHH, jj >= CHH))
          def _():
            pltpu.sync_copy(ones_v, cnt_sh.at[dst_v.at[jj]], add=True)
        bp = (b + nb - 1) % nb

        @pl.when(jnp.logical_and(jj >= 1, jj + nb - 1 < CH))
        def _():
          pltpu.make_async_copy(bufs[bp], agg_sh.at[dst_v.at[jj - 1]],
                                sem_s[bp]).wait()
          gather(jj + nb - 1, bp)
      return 0

    lax.fori_loop(0, CH // nb, step, 0)
    for b in range(nb):
      m = CH - nb + b
      pltpu.make_async_copy(bufs[m % nb], agg_sh.at[dst_v.at[m]],
                            sem_s[m % nb]).wait()
    plsc.subcore_barrier()

    row_sl = pl.ds(sid * RPS, RPS)

    @pl.when(cid == 0)
    def _():
      pltpu.sync_copy(agg_sh.at[row_sl], agg_out.at[row_sl, pl.ds(0, DH)])

    @pl.when(cid == 1)
    def _():
      pltpu.sync_copy(agg_sh.at[row_sl], agg_out.at[row_sl, pl.ds(DH, DH)])
    if with_counts:
      pltpu.sync_copy(cnt_sh.at[row_sl], cnt_out.at[cid].at[row_sl])

  return pl.kernel(body, out_type=tuple(out_type), mesh=mesh,
                   scratch_types=scratch,
                   compiler_params=pltpu.CompilerParams(
                       use_tc_tiling_on_sc=False))


_sc_agg_counts = _sc_pass(with_counts=True)
_sc_agg = _sc_pass(with_counts=False)


def _tc_combine(relu: bool):
  BLK = 2000
  GRID = N // BLK

  def body(agg, cntp, x, wlt, wrt, b, o):
    cnt = cntp[0, :, 0:1] + cntp[1, :, 0:1]
    mean = agg[...] / jnp.maximum(cnt, 1.0)
    y = (jnp.dot(mean, wlt[...], preferred_element_type=jnp.float32)
         + jnp.dot(x[...], wrt[...], preferred_element_type=jnp.float32)
         + b[...])
    o[...] = jnp.maximum(y, 0.0) if relu else y

  return pl.pallas_call(
      body,
      grid=(GRID,),
      in_specs=[
          pl.BlockSpec((BLK, D), lambda i: (i, 0)),
          pl.BlockSpec((NC, BLK, CW), lambda i: (0, i, 0)),
          pl.BlockSpec((BLK, D), lambda i: (i, 0)),
          pl.BlockSpec((D, D), lambda i: (0, 0)),
          pl.BlockSpec((D, D), lambda i: (0, 0)),
          pl.BlockSpec((1, D), lambda i: (0, 0)),
      ],
      out_specs=pl.BlockSpec((BLK, D), lambda i: (i, 0)),
      out_shape=jax.ShapeDtypeStruct((N, D), jnp.float32),
  )


_tc_combine1 = _tc_combine(relu=True)
_tc_combine2 = _tc_combine(relu=False)


def kernel(x, edge_index, W1l, b1, W1r, W2l, b2, W2r):
  pad_src = jnp.arange(EPAD, dtype=jnp.int32) % N
  src1 = jnp.concatenate([edge_index[0], pad_src])
  src = jnp.concatenate([src1 * 2, src1 * 2 + 1]).reshape(NC * NS, CH, K)
  pad_dst = N + (jnp.arange(EPAD, dtype=jnp.int32) % (NP - N))
  dst = jnp.concatenate([edge_index[1], pad_dst]).reshape(NS, CH, K)
  ones_h = jnp.ones((K, CW), jnp.float32)

  agg1, cnt = _sc_agg_counts(x.reshape(NC * N, DH), src, dst, ones_h)
  h = _tc_combine1(agg1, cnt, x, W1l.T, W1r.T, b1.reshape(1, D))
  (agg2,) = _sc_agg(h.reshape(NC * N, DH), src, dst, ones_h)
  out = _tc_combine2(agg2, cnt, h, W2l.T, W2r.T, b2.reshape(1, D))
  return out


# speedup vs baseline: 3.2756x; 1.0356x over previous
import jax
import jax.numpy as jnp
from jax import lax
from jax.experimental import pallas as pl
from jax.experimental.pallas import tpu as pltpu
from jax.experimental.pallas import tpu_sc as plsc

N = 10000
E = 320000
D = 128
DH = D // 2
NC = 2
NS = 16
K = 128
CH = 160
EPW = CH * K
EPAD = NS * EPW - E
CHH = CH // 2
NP = 10240
DUMMY = N + 16
RPS = NP // NS
CW = 16


def _sc_pass(with_counts: bool):
  nb = 4 if with_counts else 5
  out_type = [jax.ShapeDtypeStruct((NP, D), jnp.float32)]
  scratch = [
      pltpu.VMEM((CH, K), jnp.int32),
      pltpu.VMEM((CH, K), jnp.int32),
  ]
  scratch += [pltpu.VMEM((K, DH), jnp.float32) for _ in range(nb)]
  scratch += [pltpu.SemaphoreType.DMA for _ in range(2 * nb)]
  if with_counts:
    out_type.append(jax.ShapeDtypeStruct((NC, NP, CW), jnp.float32))
    scratch += [pltpu.VMEM((K, CW), jnp.float32)]
  scratch += [pltpu.VMEM_SHARED((NP, DH), jnp.float32)]
  if with_counts:
    scratch += [pltpu.VMEM_SHARED((NP, CW), jnp.float32)]

  mesh = plsc.VectorSubcoreMesh(core_axis_name="c", subcore_axis_name="s")

  def body(table, srcs, dsts, ones_h, *rest):
    if with_counts:
      agg_out, cnt_out = rest[0], rest[1]
      rest = rest[2:]
    else:
      agg_out = rest[0]
      rest = rest[1:]
    src_v, dst_v = rest[0], rest[1]
    bufs = rest[2:2 + nb]
    sem_g = rest[2 + nb:2 + 2 * nb]
    sem_s = rest[2 + 2 * nb:2 + 3 * nb]
    rest = rest[2 + 3 * nb:]
    if with_counts:
      ones_v, agg_sh, cnt_sh = rest
    else:
      (agg_sh,) = rest
    cid = lax.axis_index("c")
    sid = lax.axis_index("s")

    pltpu.sync_copy(srcs.at[sid], src_v)
    pltpu.sync_copy(dsts.at[sid], dst_v)
    if with_counts:
      pltpu.sync_copy(ones_h, ones_v)
    z16 = jnp.zeros((16,), jnp.float32)

    def zrow(r, _):
      for c in range(DH // 16):
        bufs[0][r, pl.ds(c * 16, 16)] = z16
      return 0

    lax.fori_loop(0, K, zrow, 0)
    for r in range(RPS // K):
      pltpu.sync_copy(bufs[0], agg_sh.at[pl.ds(sid * RPS + r * K, K)])
    if with_counts:
      for r in range(RPS // K):
        pltpu.sync_copy(bufs[0].at[:, pl.ds(0, CW)],
                        cnt_sh.at[pl.ds(sid * RPS + r * K, K)])
    plsc.subcore_barrier()

    tbl = table.at[pl.ds(cid, NC * N - 1)]

    def gather(jj, b):
      return pltpu.async_copy(tbl.at[src_v.at[jj]], bufs[b], sem_g[b])

    def scatter(jj, b):
      return pltpu.async_copy(bufs[b], agg_sh.at[dst_v.at[jj]], sem_s[b],
                              add=True)

    for b in range(nb):
      gather(b, b)

    def step(j, _):
      for b in range(nb):
        jj = j * nb + b
        pltpu.make_async_copy(tbl.at[src_v.at[jj]], bufs[b],
                              sem_g[b]).wait()
        scatter(jj, b)
        if with_counts:
          @pl.when(lax.select(cid == 0, jj < CHH, jj >= CHH))
          def _():
            pltpu.sync_copy(ones_v, cnt_sh.at[dst_v.at[jj]], add=True)
        bp = (b + nb - 1) % nb

        @pl.when(jnp.logical_and(jj >= 1, jj + nb - 1 < CH))
        def _():
          pltpu.make_async_copy(bufs[bp], agg_sh.at[dst_v.at[jj - 1]],
                                sem_s[bp]).wait()
          gather(jj + nb - 1, bp)
      return 0

    lax.fori_loop(0, CH // nb, step, 0)
    for b in range(nb):
      m = CH - nb + b
      pltpu.make_async_copy(bufs[m % nb], agg_sh.at[dst_v.at[m]],
                            sem_s[m % nb]).wait()
    plsc.subcore_barrier()

    row_sl = pl.ds(sid * RPS, RPS)

    @pl.when(cid == 0)
    def _():
      pltpu.sync_copy(agg_sh.at[row_sl], agg_out.at[row_sl, pl.ds(0, DH)])

    @pl.when(cid == 1)
    def _():
      pltpu.sync_copy(agg_sh.at[row_sl], agg_out.at[row_sl, pl.ds(DH, DH)])
    if with_counts:
      pltpu.sync_copy(cnt_sh.at[row_sl], cnt_out.at[cid].at[row_sl])

  return pl.kernel(body, out_type=tuple(out_type), mesh=mesh,
                   scratch_types=scratch,
                   compiler_params=pltpu.CompilerParams(
                       use_tc_tiling_on_sc=False))


_sc_agg_counts = _sc_pass(with_counts=True)
_sc_agg = _sc_pass(with_counts=False)


def _tc_combine(relu: bool):
  BLK = 2000
  GRID = N // BLK

  def body(agg, cntp, x, wlt, wrt, b, o):
    cnt = cntp[0, :, 0:1] + cntp[1, :, 0:1]
    mean = agg[...] / jnp.maximum(cnt, 1.0)
    y = (jnp.dot(mean, wlt[...], preferred_element_type=jnp.float32)
         + jnp.dot(x[...], wrt[...], preferred_element_type=jnp.float32)
         + b[...])
    o[...] = jnp.maximum(y, 0.0) if relu else y

  return pl.pallas_call(
      body,
      grid=(GRID,),
      in_specs=[
          pl.BlockSpec((BLK, D), lambda i: (i, 0)),
          pl.BlockSpec((NC, BLK, CW), lambda i: (0, i, 0)),
          pl.BlockSpec((BLK, D), lambda i: (i, 0)),
          pl.BlockSpec((D, D), lambda i: (0, 0)),
          pl.BlockSpec((D, D), lambda i: (0, 0)),
          pl.BlockSpec((1, D), lambda i: (0, 0)),
      ],
      out_specs=pl.BlockSpec((BLK, D), lambda i: (i, 0)),
      out_shape=jax.ShapeDtypeStruct((N, D), jnp.float32),
  )


_tc_combine1 = _tc_combine(relu=True)
_tc_combine2 = _tc_combine(relu=False)


def kernel(x, edge_index, W1l, b1, W1r, W2l, b2, W2r):
  pad_src = jnp.arange(EPAD, dtype=jnp.int32) % N
  src = (jnp.concatenate([edge_index[0], pad_src]) * 2).reshape(NS, CH, K)
  pad_dst = N + (jnp.arange(EPAD, dtype=jnp.int32) % (NP - N))
  dst = jnp.concatenate([edge_index[1], pad_dst]).reshape(NS, CH, K)
  ones_h = jnp.ones((K, CW), jnp.float32)

  agg1, cnt = _sc_agg_counts(x.reshape(NC * N, DH), src, dst, ones_h)
  h = _tc_combine1(agg1, cnt, x, W1l.T, W1r.T, b1.reshape(1, D))
  (agg2,) = _sc_agg(h.reshape(NC * N, DH), src, dst, ones_h)
  out = _tc_combine2(agg2, cnt, h, W2l.T, W2r.T, b2.reshape(1, D))
  return out
